# baseline (speedup 1.0000x reference)
import functools

import jax
import jax.numpy as jnp
from jax import lax
from jax.experimental import pallas as pl
from jax.experimental.pallas import tpu as pltpu
from jax.experimental.pallas import tpu_sc as plsc

N = 1000
E = 1000
IN_F = 128
N_HID = 64
OUT_F = 32
END_F = 2
DEG_SIZE = 64
LINE_LEN = 999
KNN_K = 10
FC_DIM = N * 2 * END_F
FC_BLK = 512

_NEG = -1e9


def _leaky(x):
    return jnp.where(x >= 0, x, 0.2 * x)


def _c00(a, b):
    return lax.dot_general(a, b, (((0,), (0,)), ((), ())))


def _c11(a, b):
    return lax.dot_general(a, b, (((1,), (1,)), ((), ())))


def _gat_body(adj_ref, bct_ref, wbc1_ref, bbc1_ref, wbc2_ref, bbc2_ref,
              degt_ref, wg1_ref, a1s_ref, a1d_ref, wg2_ref, a2s_ref,
              a2d_ref, wl_ref, bl_ref, x_ref, gat_ref):
    adj = adj_ref[...]
    hbc = jnp.maximum(bct_ref[...] @ wbc1_ref[...] + bbc1_ref[...], 0.0)
    bc_f = hbc @ wbc2_ref[...] + bbc2_ref[...]
    deg = jnp.clip(jnp.sum(adj, axis=1, keepdims=True).astype(jnp.int32),
                   0, DEG_SIZE - 1)
    buckets = lax.broadcasted_iota(jnp.int32, (1, DEG_SIZE), 1)
    onehot = (deg == buckets).astype(jnp.float32)
    x_deg = onehot @ degt_ref[...]
    x = jnp.concatenate([x_deg, bc_f], axis=1)
    x_ref[...] = x

    mask = adj > 0

    def gat_layer(h, a_s_row, a_d_row):
        s_col = jnp.sum(h * a_s_row, axis=1, keepdims=True)
        t_row = _c11(a_d_row, h)
        e = _leaky(s_col + t_row)
        e = jnp.where(mask, e, _NEG)
        e = e - jnp.max(e, axis=1, keepdims=True)
        p = jnp.exp(e)
        num = p @ h
        den = jnp.sum(p, axis=1, keepdims=True)
        return num / den

    h1 = x @ wg1_ref[...]
    g1 = gat_layer(h1, a1s_ref[...], a1d_ref[...])
    g1 = jnp.where(g1 > 0, g1, jnp.exp(g1) - 1.0)
    h2 = g1 @ wg2_ref[...]
    g2 = gat_layer(h2, a2s_ref[...], a2d_ref[...])
    gat_ref[...] = g2 @ wl_ref[...] + bl_ref[...]


def _hyper_body(hga_ref, x_ref, wh1_ref, an1_ref, ae1_ref, wh2_ref,
                an2_ref, ae2_ref, wl2_ref, bl2_ref, emb_ref, emb0_ref):
    mask_t = hga_ref[...] > 0
    ones_col_e = jnp.ones((E, 1), jnp.float32)
    ones_col_n = jnp.ones((N, 1), jnp.float32)

    def layer(x1, an_row, ae_row, edge_contrib):
        s_n_row = _leaky(_c11(an_row, x1))
        le = jnp.where(mask_t, s_n_row, _NEG)
        le = le - jnp.max(le, axis=1, keepdims=True)
        pe = jnp.exp(le)
        ef = (pe @ x1) / (pe @ ones_col_n)
        if edge_contrib is not None:
            ef = ef + edge_contrib
        s_e_col = _leaky(jnp.sum(ef * ae_row, axis=1, keepdims=True))
        ln = jnp.where(mask_t, s_e_col, _NEG)
        ln = ln - jnp.max(ln, axis=0, keepdims=True)
        q = jnp.exp(ln)
        node = _c00(q, ef) / _c00(q, ones_col_e)
        return node, ef

    x1a = x_ref[...] @ wh1_ref[...]
    n1, ef1 = layer(x1a, an1_ref[...], ae1_ref[...], None)
    x1b = n1 @ wh2_ref[...]
    n2, _ = layer(x1b, an2_ref[...], ae2_ref[...], ef1 @ wh2_ref[...])
    emb_ref[...] = n2
    emb0_ref[...] = n2 @ wl2_ref[...] + bl2_ref[...]


def _knn_prep_body(xemb_ref, w31_ref, a31n_ref, nn_ref, pay_ref, mean_ref):
    xe = xemb_ref[...]
    sq = xe * xe
    d_col = jnp.sum(sq, axis=1, keepdims=True)
    d_row = _c11(jnp.ones((1, OUT_F), jnp.float32), sq)
    g = _c11(xe, xe)
    v = -(d_col + d_row - 2.0 * g)
    sent = -3e38
    jidx = lax.broadcasted_iota(jnp.int32, (N, N), 1)
    cols = []
    for _ in range(KNN_K):
        m = jnp.max(v, axis=1, keepdims=True)
        cand = jnp.where(v == m, jidx, N)
        jstar = jnp.min(cand, axis=1, keepdims=True)
        cols.append(jstar)
        v = jnp.where(jidx == jstar, sent, v)
    nn_ref[...] = jnp.concatenate(
        cols + [jnp.zeros((N, 6), jnp.int32)], axis=1)

    x3 = xe @ w31_ref[...]
    s_col = _leaky(jnp.sum(x3 * a31n_ref[...], axis=1, keepdims=True))
    w = jnp.exp(s_col - jnp.max(s_col))
    pay_ref[...] = jnp.concatenate(
        [w * x3, w, jnp.zeros((N, 10), jnp.float32)], axis=1)
    mean_ref[...] = jnp.concatenate(
        [jnp.mean(x3, axis=0, keepdims=True),
         jnp.zeros((1, 11), jnp.float32)], axis=1)


_SC_NP = 64
_SC_W = 16


_SC_PAD = 16 * _SC_NP


def _sc_hyper34_body(pairsf_h, payc_h, consts_h, out_h,
                     pairsf_v, payc_v, colb_v, e3loc_v, loc_v,
                     pay4_v, out_v, stg_v, stgm_v, consts_v,
                     s3, e3, s4, e4, stgs, sem):
    s = lax.axis_index("s")
    base = s * _SC_NP
    D = pl.ds

    def leaky(x):
        return jnp.where(x >= 0, x, 0.2 * x)

    def crow(r):
        return consts_v[D(r * 16, 16)]

    pltpu.sync_copy(pairsf_h.at[s], pairsf_v)
    pltpu.sync_copy(payc_h.at[s], payc_v)
    pltpu.sync_copy(consts_h, consts_v)
    zeros16 = jnp.zeros((16,), jnp.float32)
    for g in range(8):
        out_v[D(g * 16, 16)] = zeros16
    for c in range(6):
        pltpu.sync_copy(out_v.at[D(0, _SC_NP)], s3[c].at[D(base, _SC_NP)])
    for c in range(2):
        pltpu.sync_copy(out_v.at[D(0, _SC_NP)], s4[c].at[D(base, _SC_NP)])
    plsc.subcore_barrier()

    for k in range(KNN_K):
        descs = []
        for g in range(4):
            ix = pairsf_v[D(k * _SC_NP + g * 16, 16)]
            for c in range(6):
                descs.append(pltpu.async_copy(
                    payc_v.at[D(c * _SC_NP + g * 16, 16)],
                    s3[c].at[ix], sem, add=True))
        for d in descs:
            d.wait()
    plsc.subcore_barrier()

    for c in range(6):
        pltpu.sync_copy(s3[c].at[D(base, _SC_NP)],
                        colb_v.at[D(c * _SC_NP, _SC_NP)])
    for g in range(4):
        den = colb_v[D(5 * _SC_NP + g * 16, 16)]
        has = den > 0.0
        efs = []
        for o in range(5):
            num = colb_v[D(o * _SC_NP + g * 16, 16)]
            ef = jnp.where(has, num / den, crow(12 + o))
            efs.append(ef)
            e3loc_v[D(o * _SC_NP + g * 16, 16)] = ef
        se = efs[0] * crow(0)
        fw = efs[0] * crow(5)
        for o in range(1, 5):
            se = se + efs[o] * crow(o)
            fw = fw + efs[o] * crow(5 + o)
        e3loc_v[D(5 * _SC_NP + g * 16, 16)] = leaky(se)
        e3loc_v[D(6 * _SC_NP + g * 16, 16)] = fw
    for c in range(6):
        pltpu.sync_copy(e3loc_v.at[D(c * _SC_NP, _SC_NP)],
                        e3[c].at[D(base, _SC_NP)])
    plsc.subcore_barrier()

    for c in range(6):
        pltpu.sync_copy(e3[c], loc_v.at[D(c * _SC_PAD, _SC_PAD)])
    xsum = jnp.zeros((16,), jnp.float32)
    for g in range(4):
        idxs = [pairsf_v[D(k * _SC_NP + g * 16, 16)] for k in range(KNN_K)]
        svals = [plsc.load_gather(loc_v, [5 * _SC_PAD + ix]) for ix in idxs]
        m = svals[0]
        for k in range(1, KNN_K):
            m = jnp.maximum(m, svals[k])
        ps = [jnp.exp(sv - m) for sv in svals]
        z = ps[0]
        for k in range(1, KNN_K):
            z = z + ps[k]
        x4 = jnp.zeros((16,), jnp.float32)
        for o in range(5):
            acc = ps[0] * plsc.load_gather(loc_v, [o * _SC_PAD + idxs[0]])
            for k in range(1, KNN_K):
                acc = acc + ps[k] * plsc.load_gather(
                    loc_v, [o * _SC_PAD + idxs[k]])
            x4 = x4 + (acc / z) * crow(5 + o)
        w4 = jnp.exp(leaky(x4 * crow(10)))
        valid = (base + g * 16 + lax.iota(jnp.int32, 16)) < N
        w4m = jnp.where(valid, w4, 0.0)
        pay4_v[D(g * 16, 16)] = w4m * x4
        pay4_v[D(_SC_NP + g * 16, 16)] = w4m
        xsum = xsum + jnp.where(valid, x4, 0.0)
    stg_v[...] = xsum
    pltpu.sync_copy(stg_v, stgs.at[D(s * 16, 16)])

    for k in range(KNN_K):
        descs = []
        for g in range(4):
            ix = pairsf_v[D(k * _SC_NP + g * 16, 16)]
            for c in range(2):
                descs.append(pltpu.async_copy(
                    pay4_v.at[D(c * _SC_NP + g * 16, 16)],
                    s4[c].at[ix], sem, add=True))
        for d in descs:
            d.wait()
    plsc.subcore_barrier()

    pltpu.sync_copy(stgs, stgm_v)
    tot = stgm_v[D(0, 16)]
    for si in range(1, 16):
        tot = tot + stgm_v[D(si * 16, 16)]
    mean4 = jnp.broadcast_to(jnp.sum(tot * (1.0 / N)), (16,))
    for c in range(2):
        pltpu.sync_copy(s4[c].at[D(base, _SC_NP)],
                        colb_v.at[D(c * _SC_NP, _SC_NP)])
    for g in range(4):
        den = colb_v[D(_SC_NP + g * 16, 16)]
        num = colb_v[D(g * 16, 16)]
        x1e = jnp.where(den > 0.0, num / den, mean4)
        efv = x1e + e3loc_v[D(6 * _SC_NP + g * 16, 16)]
        pay4_v[D(g * 16, 16)] = efv
        pay4_v[D(_SC_NP + g * 16, 16)] = leaky(efv * crow(11))
    for c in range(2):
        pltpu.sync_copy(pay4_v.at[D(c * _SC_NP, _SC_NP)],
                        e4[c].at[D(base, _SC_NP)])
    plsc.subcore_barrier()

    for c in range(2):
        pltpu.sync_copy(e4[c], loc_v.at[D(c * _SC_PAD, _SC_PAD)])
    for g in range(4):
        idxs = [pairsf_v[D(k * _SC_NP + g * 16, 16)] for k in range(KNN_K)]
        svals = [plsc.load_gather(loc_v, [_SC_PAD + ix]) for ix in idxs]
        m = svals[0]
        for k in range(1, KNN_K):
            m = jnp.maximum(m, svals[k])
        ps = [jnp.exp(sv - m) for sv in svals]
        z = ps[0]
        for k in range(1, KNN_K):
            z = z + ps[k]
        acc = ps[0] * plsc.load_gather(loc_v, [idxs[0]])
        for k in range(1, KNN_K):
            acc = acc + ps[k] * plsc.load_gather(loc_v, [idxs[k]])
        out_v[D(g * 16, 16)] = acc / z
    pltpu.sync_copy(out_v, out_h.at[s])


def _sc_hyper34(pairsf, payc, consts):
    mesh = plsc.VectorSubcoreMesh(core_axis_name="c", subcore_axis_name="s",
                                  num_cores=1)
    shared1d = pltpu.VMEM_SHARED((_SC_PAD,), jnp.float32)
    fn = pl.kernel(
        _sc_hyper34_body,
        out_type=jax.ShapeDtypeStruct((16, 128), jnp.float32),
        mesh=mesh,
        compiler_params=pltpu.CompilerParams(needs_layout_passes=False),
        scratch_types=[
            pltpu.VMEM((KNN_K * _SC_NP,), jnp.int32),
            pltpu.VMEM((6 * _SC_NP,), jnp.float32),
            pltpu.VMEM((6 * _SC_NP,), jnp.float32),
            pltpu.VMEM((7 * _SC_NP,), jnp.float32),
            pltpu.VMEM((6 * _SC_PAD,), jnp.float32),
            pltpu.VMEM((2 * _SC_NP,), jnp.float32),
            pltpu.VMEM((128,), jnp.float32),
            pltpu.VMEM((16,), jnp.float32),
            pltpu.VMEM((256,), jnp.float32),
            pltpu.VMEM((512,), jnp.float32),
            [shared1d] * 6,
            [shared1d] * 6,
            [shared1d] * 2,
            [shared1d] * 2,
            pltpu.VMEM_SHARED((256,), jnp.float32),
            pltpu.SemaphoreType.DMA,
        ],
    )
    return fn(pairsf, payc, consts)


def _head_body(emb_ref, wfc_ref, bfc_ref, wfc3_ref, bfc3_ref, out_ref):
    j = pl.program_id(0)

    @pl.when(j == 0)
    def _():
        out_ref[...] = jnp.zeros_like(out_ref)

    h = emb_ref[...] @ wfc_ref[...] + bfc_ref[...]
    h = jnp.where(h >= 0, h, 0.01 * h)
    col = j * FC_BLK + lax.broadcasted_iota(jnp.int32, (1, FC_BLK), 1)
    h = jnp.where(col < FC_DIM, h, 0.0)
    w3 = wfc3_ref[...]
    rowi = j * FC_BLK + lax.broadcasted_iota(jnp.int32, (FC_BLK, 1), 0)
    w3 = jnp.where(rowi < FC_DIM, w3, 0.0)
    out_ref[...] += h @ w3

    @pl.when(j == pl.num_programs(0) - 1)
    def _():
        out_ref[...] = jax.nn.sigmoid(out_ref[...] + bfc3_ref[...])


def _vmem_params():
    return pltpu.CompilerParams(vmem_limit_bytes=100 * 1024 * 1024)


def kernel(A, hypergraph_adj, adj, hypergraph_khop_and_k_shell, BC, params):
    p = params
    row = lambda v: v.reshape(1, -1)

    x, gat_out = pl.pallas_call(
        _gat_body,
        out_shape=[jax.ShapeDtypeStruct((N, IN_F), jnp.float32),
                   jax.ShapeDtypeStruct((N, 1), jnp.float32)],
        compiler_params=_vmem_params(),
    )(adj, BC.T, p["W_bc1"], row(p["b_bc1"]), p["W_bc2"], row(p["b_bc2"]),
      p["deg_table"], p["Wg1"], row(p["a1s"]), row(p["a1d"]), p["Wg2"],
      row(p["a2s"]), row(p["a2d"]), p["Wl"], row(p["bl"]))

    hyp_emb, emb0 = pl.pallas_call(
        _hyper_body,
        out_shape=[jax.ShapeDtypeStruct((N, OUT_F), jnp.float32),
                   jax.ShapeDtypeStruct((N, END_F), jnp.float32)],
        compiler_params=_vmem_params(),
    )(hypergraph_adj, x, p["Wh1"], row(p["an1"]), row(p["ae1"]), p["Wh2"],
      row(p["an2"]), row(p["ae2"]), p["Wl2"], row(p["bl2"]))

    nn16, payload, meanrow = pl.pallas_call(
        _knn_prep_body,
        out_shape=[jax.ShapeDtypeStruct((N, 16), jnp.int32),
                   jax.ShapeDtypeStruct((N, _SC_W), jnp.float32),
                   jax.ShapeDtypeStruct((1, _SC_W), jnp.float32)],
        compiler_params=_vmem_params(),
    )(hyp_emb, p["W31"], row(p["a31n"]))

    nn10 = jnp.concatenate(
        [nn16[:, :KNN_K], jnp.zeros((_SC_PAD - N, KNN_K), jnp.int32)], axis=0)
    pairsf = nn10.reshape(16, _SC_NP, KNN_K).transpose(0, 2, 1).reshape(
        16, KNN_K * _SC_NP)
    payp = jnp.concatenate(
        [payload[:, :6], jnp.zeros((_SC_PAD - N, 6), jnp.float32)], axis=0)
    payc = payp.reshape(16, _SC_NP, 6).transpose(0, 2, 1).reshape(
        16, 6 * _SC_NP)
    cvec = jnp.concatenate(
        [p["a31e"], p["W32"][:, 0], p["a32n"], p["a32e"], meanrow[0, :5],
         jnp.zeros((15,), jnp.float32)])
    consts = jnp.tile(cvec[:, None], (1, 16)).reshape(512)
    hyp3 = _sc_hyper34(pairsf, payc, consts)[:, :_SC_NP].reshape(
        _SC_PAD)[:N].reshape(N, 1)

    emb = jnp.concatenate([gat_out, emb0, hyp3], axis=1).reshape(1, -1)

    nblk = (FC_DIM + FC_BLK - 1) // FC_BLK
    out = pl.pallas_call(
        _head_body,
        grid=(nblk,),
        in_specs=[
            pl.BlockSpec((1, FC_DIM), lambda j: (0, 0)),
            pl.BlockSpec((FC_DIM, FC_BLK), lambda j: (0, j)),
            pl.BlockSpec((1, FC_BLK), lambda j: (0, j)),
            pl.BlockSpec((FC_BLK, LINE_LEN), lambda j: (j, 0)),
            pl.BlockSpec((1, LINE_LEN), lambda j: (0, 0)),
        ],
        out_specs=pl.BlockSpec((1, LINE_LEN), lambda j: (0, 0)),
        out_shape=jax.ShapeDtypeStruct((1, LINE_LEN), jnp.float32),
        compiler_params=_vmem_params(),
    )(emb, p["Wfc"], row(p["bfc"]), p["Wfc3"], row(p["bfc3"]))

    return out


# speedup vs baseline: 1.0637x; 1.0637x over previous
import functools

import jax
import jax.numpy as jnp
from jax import lax
from jax.experimental import pallas as pl
from jax.experimental.pallas import tpu as pltpu
from jax.experimental.pallas import tpu_sc as plsc

N = 1000
E = 1000
IN_F = 128
N_HID = 64
OUT_F = 32
END_F = 2
DEG_SIZE = 64
LINE_LEN = 999
KNN_K = 10
FC_DIM = N * 2 * END_F
FC_BLK = 512

_NEG = -1e9


def _leaky(x):
    return jnp.where(x >= 0, x, 0.2 * x)


def _c00(a, b):
    return lax.dot_general(a, b, (((0,), (0,)), ((), ())))


def _c11(a, b):
    return lax.dot_general(a, b, (((1,), (1,)), ((), ())))


def _gat_body(adj_ref, bct_ref, wbc1_ref, bbc1_ref, wbc2_ref, bbc2_ref,
              degt_ref, wg1_ref, a1s_ref, a1d_ref, wg2_ref, a2s_ref,
              a2d_ref, wl_ref, bl_ref, x_ref, gat_ref):
    adj = adj_ref[...]
    hbc = jnp.maximum(bct_ref[...] @ wbc1_ref[...] + bbc1_ref[...], 0.0)
    bc_f = hbc @ wbc2_ref[...] + bbc2_ref[...]
    deg = jnp.clip(jnp.sum(adj, axis=1, keepdims=True).astype(jnp.int32),
                   0, DEG_SIZE - 1)
    buckets = lax.broadcasted_iota(jnp.int32, (1, DEG_SIZE), 1)
    onehot = (deg == buckets).astype(jnp.float32)
    x_deg = onehot @ degt_ref[...]
    x = jnp.concatenate([x_deg, bc_f], axis=1)
    x_ref[...] = x

    mask = adj > 0

    def gat_layer(h, a_s_row, a_d_row):
        s_col = jnp.sum(h * a_s_row, axis=1, keepdims=True)
        t_row = _c11(a_d_row, h)
        e = _leaky(s_col + t_row)
        e = jnp.where(mask, e, _NEG)
        e = e - jnp.max(e, axis=1, keepdims=True)
        p = jnp.exp(e)
        num = p @ h
        den = jnp.sum(p, axis=1, keepdims=True)
        return num / den

    h1 = x @ wg1_ref[...]
    g1 = gat_layer(h1, a1s_ref[...], a1d_ref[...])
    g1 = jnp.where(g1 > 0, g1, jnp.exp(g1) - 1.0)
    h2 = g1 @ wg2_ref[...]
    g2 = gat_layer(h2, a2s_ref[...], a2d_ref[...])
    gat_ref[...] = g2 @ wl_ref[...] + bl_ref[...]


def _hyper_body(hga_ref, x_ref, wh1_ref, an1_ref, ae1_ref, wh2_ref,
                an2_ref, ae2_ref, wl2_ref, bl2_ref, emb_ref, emb0_ref):
    mask_t = hga_ref[...] > 0
    ones_col_e = jnp.ones((E, 1), jnp.float32)
    ones_col_n = jnp.ones((N, 1), jnp.float32)

    def layer(x1, an_row, ae_row, edge_contrib):
        s_n_row = _leaky(_c11(an_row, x1))
        le = jnp.where(mask_t, s_n_row, _NEG)
        le = le - jnp.max(le, axis=1, keepdims=True)
        pe = jnp.exp(le)
        ef = (pe @ x1) / (pe @ ones_col_n)
        if edge_contrib is not None:
            ef = ef + edge_contrib
        s_e_col = _leaky(jnp.sum(ef * ae_row, axis=1, keepdims=True))
        ln = jnp.where(mask_t, s_e_col, _NEG)
        ln = ln - jnp.max(ln, axis=0, keepdims=True)
        q = jnp.exp(ln)
        node = _c00(q, ef) / _c00(q, ones_col_e)
        return node, ef

    x1a = x_ref[...] @ wh1_ref[...]
    n1, ef1 = layer(x1a, an1_ref[...], ae1_ref[...], None)
    x1b = n1 @ wh2_ref[...]
    n2, _ = layer(x1b, an2_ref[...], ae2_ref[...], ef1 @ wh2_ref[...])
    emb_ref[...] = n2
    emb0_ref[...] = n2 @ wl2_ref[...] + bl2_ref[...]


def _knn_prep_body(xemb_ref, w31_ref, a31n_ref, nn_ref, pay_ref, mean_ref):
    xe = xemb_ref[...]
    sq = xe * xe
    d_col = jnp.sum(sq, axis=1, keepdims=True)
    d_row = _c11(jnp.ones((1, OUT_F), jnp.float32), sq)
    g = _c11(xe, xe)
    v = -(d_col + d_row - 2.0 * g)
    sent = -3e38
    jidx = lax.broadcasted_iota(jnp.int32, (N, N), 1)
    cols = []
    for _ in range(KNN_K):
        m = jnp.max(v, axis=1, keepdims=True)
        cand = jnp.where(v == m, jidx, N)
        jstar = jnp.min(cand, axis=1, keepdims=True)
        cols.append(jstar)
        v = jnp.where(jidx == jstar, sent, v)
    nn_ref[...] = jnp.concatenate(
        cols + [jnp.zeros((N, 6), jnp.int32)], axis=1)

    x3 = xe @ w31_ref[...]
    s_col = _leaky(jnp.sum(x3 * a31n_ref[...], axis=1, keepdims=True))
    w = jnp.exp(s_col - jnp.max(s_col))
    pay_ref[...] = jnp.concatenate(
        [w * x3, w, jnp.zeros((N, 10), jnp.float32)], axis=1)
    mean_ref[...] = jnp.concatenate(
        [jnp.mean(x3, axis=0, keepdims=True),
         jnp.zeros((1, 11), jnp.float32)], axis=1)


_SC_NP = 64
_SC_W = 16


_SC_PAD = 16 * _SC_NP


def _sc_hyper34_body(pairsf_h, payc_h, consts_h, out_h,
                     pairsf_v, payc_v, colb_v, e3loc_v, loc_v,
                     pay4_v, out_v, stg_v, stgm_v, consts_v,
                     s3, e3, s4, e4, stgs, sem):
    s = lax.axis_index("s")
    base = s * _SC_NP
    D = pl.ds

    def leaky(x):
        return jnp.where(x >= 0, x, 0.2 * x)

    def crow(r):
        return consts_v[D(r * 16, 16)]

    pltpu.sync_copy(pairsf_h.at[s], pairsf_v)
    pltpu.sync_copy(payc_h.at[s], payc_v)
    pltpu.sync_copy(consts_h, consts_v)
    zeros16 = jnp.zeros((16,), jnp.float32)
    for g in range(8):
        out_v[D(g * 16, 16)] = zeros16
    for c in range(6):
        pltpu.sync_copy(out_v.at[D(0, _SC_NP)], s3[c].at[D(base, _SC_NP)])
    for c in range(2):
        pltpu.sync_copy(out_v.at[D(0, _SC_NP)], s4[c].at[D(base, _SC_NP)])
    plsc.subcore_barrier()

    descs = []
    for k in range(KNN_K):
        for g in range(4):
            ix = pairsf_v[D(k * _SC_NP + g * 16, 16)]
            for c in range(6):
                descs.append(pltpu.async_copy(
                    payc_v.at[D(c * _SC_NP + g * 16, 16)],
                    s3[c].at[ix], sem, add=True))
    for d in descs:
        d.wait()
    plsc.subcore_barrier()

    for c in range(6):
        pltpu.sync_copy(s3[c].at[D(base, _SC_NP)],
                        colb_v.at[D(c * _SC_NP, _SC_NP)])
    for g in range(4):
        den = colb_v[D(5 * _SC_NP + g * 16, 16)]
        has = den > 0.0
        efs = []
        for o in range(5):
            num = colb_v[D(o * _SC_NP + g * 16, 16)]
            ef = jnp.where(has, num / den, crow(12 + o))
            efs.append(ef)
            e3loc_v[D(o * _SC_NP + g * 16, 16)] = ef
        se = efs[0] * crow(0)
        fw = efs[0] * crow(5)
        for o in range(1, 5):
            se = se + efs[o] * crow(o)
            fw = fw + efs[o] * crow(5 + o)
        e3loc_v[D(5 * _SC_NP + g * 16, 16)] = leaky(se)
        e3loc_v[D(6 * _SC_NP + g * 16, 16)] = fw
    for c in range(6):
        pltpu.sync_copy(e3loc_v.at[D(c * _SC_NP, _SC_NP)],
                        e3[c].at[D(base, _SC_NP)])
    plsc.subcore_barrier()

    for c in range(6):
        pltpu.sync_copy(e3[c], loc_v.at[D(c * _SC_PAD, _SC_PAD)])
    xsum = jnp.zeros((16,), jnp.float32)
    for g in range(4):
        idxs = [pairsf_v[D(k * _SC_NP + g * 16, 16)] for k in range(KNN_K)]
        svals = [plsc.load_gather(loc_v, [5 * _SC_PAD + ix]) for ix in idxs]
        m = svals[0]
        for k in range(1, KNN_K):
            m = jnp.maximum(m, svals[k])
        ps = [jnp.exp(sv - m) for sv in svals]
        z = ps[0]
        for k in range(1, KNN_K):
            z = z + ps[k]
        x4 = jnp.zeros((16,), jnp.float32)
        for o in range(5):
            acc = ps[0] * plsc.load_gather(loc_v, [o * _SC_PAD + idxs[0]])
            for k in range(1, KNN_K):
                acc = acc + ps[k] * plsc.load_gather(
                    loc_v, [o * _SC_PAD + idxs[k]])
            x4 = x4 + (acc / z) * crow(5 + o)
        w4 = jnp.exp(leaky(x4 * crow(10)))
        valid = (base + g * 16 + lax.iota(jnp.int32, 16)) < N
        w4m = jnp.where(valid, w4, 0.0)
        pay4_v[D(g * 16, 16)] = w4m * x4
        pay4_v[D(_SC_NP + g * 16, 16)] = w4m
        xsum = xsum + jnp.where(valid, x4, 0.0)
    stg_v[...] = xsum
    pltpu.sync_copy(stg_v, stgs.at[D(s * 16, 16)])

    descs = []
    for k in range(KNN_K):
        for g in range(4):
            ix = pairsf_v[D(k * _SC_NP + g * 16, 16)]
            for c in range(2):
                descs.append(pltpu.async_copy(
                    pay4_v.at[D(c * _SC_NP + g * 16, 16)],
                    s4[c].at[ix], sem, add=True))
    for d in descs:
        d.wait()
    plsc.subcore_barrier()

    pltpu.sync_copy(stgs, stgm_v)
    tot = stgm_v[D(0, 16)]
    for si in range(1, 16):
        tot = tot + stgm_v[D(si * 16, 16)]
    mean4 = jnp.broadcast_to(jnp.sum(tot * (1.0 / N)), (16,))
    for c in range(2):
        pltpu.sync_copy(s4[c].at[D(base, _SC_NP)],
                        colb_v.at[D(c * _SC_NP, _SC_NP)])
    for g in range(4):
        den = colb_v[D(_SC_NP + g * 16, 16)]
        num = colb_v[D(g * 16, 16)]
        x1e = jnp.where(den > 0.0, num / den, mean4)
        efv = x1e + e3loc_v[D(6 * _SC_NP + g * 16, 16)]
        pay4_v[D(g * 16, 16)] = efv
        pay4_v[D(_SC_NP + g * 16, 16)] = leaky(efv * crow(11))
    for c in range(2):
        pltpu.sync_copy(pay4_v.at[D(c * _SC_NP, _SC_NP)],
                        e4[c].at[D(base, _SC_NP)])
    plsc.subcore_barrier()

    for c in range(2):
        pltpu.sync_copy(e4[c], loc_v.at[D(c * _SC_PAD, _SC_PAD)])
    for g in range(4):
        idxs = [pairsf_v[D(k * _SC_NP + g * 16, 16)] for k in range(KNN_K)]
        svals = [plsc.load_gather(loc_v, [_SC_PAD + ix]) for ix in idxs]
        m = svals[0]
        for k in range(1, KNN_K):
            m = jnp.maximum(m, svals[k])
        ps = [jnp.exp(sv - m) for sv in svals]
        z = ps[0]
        for k in range(1, KNN_K):
            z = z + ps[k]
        acc = ps[0] * plsc.load_gather(loc_v, [idxs[0]])
        for k in range(1, KNN_K):
            acc = acc + ps[k] * plsc.load_gather(loc_v, [idxs[k]])
        out_v[D(g * 16, 16)] = acc / z
    pltpu.sync_copy(out_v, out_h.at[s])


def _sc_hyper34(pairsf, payc, consts):
    mesh = plsc.VectorSubcoreMesh(core_axis_name="c", subcore_axis_name="s",
                                  num_cores=1)
    shared1d = pltpu.VMEM_SHARED((_SC_PAD,), jnp.float32)
    fn = pl.kernel(
        _sc_hyper34_body,
        out_type=jax.ShapeDtypeStruct((16, 128), jnp.float32),
        mesh=mesh,
        compiler_params=pltpu.CompilerParams(needs_layout_passes=False),
        scratch_types=[
            pltpu.VMEM((KNN_K * _SC_NP,), jnp.int32),
            pltpu.VMEM((6 * _SC_NP,), jnp.float32),
            pltpu.VMEM((6 * _SC_NP,), jnp.float32),
            pltpu.VMEM((7 * _SC_NP,), jnp.float32),
            pltpu.VMEM((6 * _SC_PAD,), jnp.float32),
            pltpu.VMEM((2 * _SC_NP,), jnp.float32),
            pltpu.VMEM((128,), jnp.float32),
            pltpu.VMEM((16,), jnp.float32),
            pltpu.VMEM((256,), jnp.float32),
            pltpu.VMEM((512,), jnp.float32),
            [shared1d] * 6,
            [shared1d] * 6,
            [shared1d] * 2,
            [shared1d] * 2,
            pltpu.VMEM_SHARED((256,), jnp.float32),
            pltpu.SemaphoreType.DMA,
        ],
    )
    return fn(pairsf, payc, consts)


def _head1_body(emb_ref, wfc_ref, bfc_ref, hacc_ref, m3_ref):
    w = wfc_ref[...]
    hacc_ref[...] = emb_ref[...] @ w + bfc_ref[...]
    m3_ref[...] = w.reshape(N, 4, FC_BLK)[:, 3, :]


def _head2_body(c3_ref, m3_ref, hacc_ref, wfc3_ref, bfc3_ref, out_ref):
    h = hacc_ref[...] + c3_ref[...] @ m3_ref[...]
    h = jnp.where(h >= 0, h, 0.01 * h)
    out_ref[...] = jax.nn.sigmoid(h @ wfc3_ref[...] + bfc3_ref[...])


def _vmem_params():
    return pltpu.CompilerParams(vmem_limit_bytes=100 * 1024 * 1024)


def kernel(A, hypergraph_adj, adj, hypergraph_khop_and_k_shell, BC, params):
    p = params
    row = lambda v: v.reshape(1, -1)

    x, gat_out = pl.pallas_call(
        _gat_body,
        out_shape=[jax.ShapeDtypeStruct((N, IN_F), jnp.float32),
                   jax.ShapeDtypeStruct((N, 1), jnp.float32)],
        compiler_params=_vmem_params(),
    )(adj, BC.T, p["W_bc1"], row(p["b_bc1"]), p["W_bc2"], row(p["b_bc2"]),
      p["deg_table"], p["Wg1"], row(p["a1s"]), row(p["a1d"]), p["Wg2"],
      row(p["a2s"]), row(p["a2d"]), p["Wl"], row(p["bl"]))

    hyp_emb, emb0 = pl.pallas_call(
        _hyper_body,
        out_shape=[jax.ShapeDtypeStruct((N, OUT_F), jnp.float32),
                   jax.ShapeDtypeStruct((N, END_F), jnp.float32)],
        compiler_params=_vmem_params(),
    )(hypergraph_adj, x, p["Wh1"], row(p["an1"]), row(p["ae1"]), p["Wh2"],
      row(p["an2"]), row(p["ae2"]), p["Wl2"], row(p["bl2"]))

    nn16, payload, meanrow = pl.pallas_call(
        _knn_prep_body,
        out_shape=[jax.ShapeDtypeStruct((N, 16), jnp.int32),
                   jax.ShapeDtypeStruct((N, _SC_W), jnp.float32),
                   jax.ShapeDtypeStruct((1, _SC_W), jnp.float32)],
        compiler_params=_vmem_params(),
    )(hyp_emb, p["W31"], row(p["a31n"]))

    nn10 = jnp.concatenate(
        [nn16[:, :KNN_K], jnp.zeros((_SC_PAD - N, KNN_K), jnp.int32)], axis=0)
    pairsf = nn10.reshape(16, _SC_NP, KNN_K).transpose(0, 2, 1).reshape(
        16, KNN_K * _SC_NP)
    payp = jnp.concatenate(
        [payload[:, :6], jnp.zeros((_SC_PAD - N, 6), jnp.float32)], axis=0)
    payc = payp.reshape(16, _SC_NP, 6).transpose(0, 2, 1).reshape(
        16, 6 * _SC_NP)
    cvec = jnp.concatenate(
        [p["a31e"], p["W32"][:, 0], p["a32n"], p["a32e"], meanrow[0, :5],
         jnp.zeros((15,), jnp.float32)])
    consts = jnp.tile(cvec[:, None], (1, 16)).reshape(512)
    hyp3 = _sc_hyper34(pairsf, payc, consts)[:, :_SC_NP].reshape(
        _SC_PAD)[:N].reshape(N, 1)

    nblk = (FC_DIM + FC_BLK - 1) // FC_BLK
    emb_abc = jnp.concatenate(
        [gat_out, emb0, jnp.zeros((N, 1), jnp.float32)],
        axis=1).reshape(1, FC_DIM)
    hacc, m3 = pl.pallas_call(
        _head1_body,
        grid=(nblk,),
        in_specs=[
            pl.BlockSpec((1, FC_DIM), lambda j: (0, 0)),
            pl.BlockSpec((FC_DIM, FC_BLK), lambda j: (0, j)),
            pl.BlockSpec((1, FC_BLK), lambda j: (0, j)),
        ],
        out_specs=[pl.BlockSpec((1, FC_BLK), lambda j: (0, j)),
                   pl.BlockSpec((N, FC_BLK), lambda j: (0, j))],
        out_shape=[jax.ShapeDtypeStruct((1, FC_DIM), jnp.float32),
                   jax.ShapeDtypeStruct((N, FC_DIM), jnp.float32)],
        compiler_params=_vmem_params(),
    )(emb_abc, p["Wfc"], row(p["bfc"]))

    out = pl.pallas_call(
        _head2_body,
        grid=(2,),
        in_specs=[
            pl.BlockSpec((1, N), lambda j: (0, 0)),
            pl.BlockSpec((N, FC_DIM), lambda j: (0, 0)),
            pl.BlockSpec((1, FC_DIM), lambda j: (0, 0)),
            pl.BlockSpec((FC_DIM, FC_BLK), lambda j: (0, j)),
            pl.BlockSpec((1, FC_BLK), lambda j: (0, j)),
        ],
        out_specs=pl.BlockSpec((1, FC_BLK), lambda j: (0, j)),
        out_shape=jax.ShapeDtypeStruct((1, LINE_LEN), jnp.float32),
        compiler_params=_vmem_params(),
    )(hyp3.T, m3, hacc, p["Wfc3"], row(p["bfc3"]))

    return out
